# baseline (speedup 1.0000x reference)
import functools

import jax
import jax.numpy as jnp
from jax import lax
from jax.experimental import pallas as pl
from jax.experimental.pallas import tpu as pltpu
from jax.experimental.pallas import tpu_sc as plsc

B = 16384
D = 64
NC = 2
NS = 16
NW = NC * NS
BPW = B // NW


def _gather6_body(h_i, r_i, t_i, he_i, re_i, te_i, ent, rel,
                  o0, o1, o2, o3, o4, o5,
                  idx_v, rows_v, sem):
    wid = lax.axis_index("s") * NC + lax.axis_index("c")
    base = wid * BPW
    tasks = ((h_i, ent, o0), (r_i, rel, o1), (t_i, ent, o2),
             (he_i, ent, o3), (re_i, rel, o4), (te_i, ent, o5))
    for idx_hbm, table, out_hbm in tasks:
        pltpu.sync_copy(idx_hbm.at[pl.ds(base, BPW)], idx_v)
        pltpu.async_copy(table.at[idx_v], rows_v, sem).wait()
        pltpu.sync_copy(rows_v, out_hbm.at[pl.ds(base, BPW)])


_mesh = plsc.VectorSubcoreMesh(core_axis_name="c", subcore_axis_name="s")

_gather6 = pl.kernel(
    _gather6_body,
    mesh=_mesh,
    out_type=tuple(jax.ShapeDtypeStruct((B, D), jnp.float32) for _ in range(6)),
    scratch_types=[
        pltpu.VMEM((BPW,), jnp.int32),
        pltpu.VMEM((BPW, D), jnp.float32),
        pltpu.SemaphoreType.DMA,
    ],
    compiler_params=pltpu.CompilerParams(use_tc_tiling_on_sc=False),
)


def kernel(pos_head, pos_rel, pos_tail, pos_head_exp, pos_rel_exp,
           pos_tail_exp, entity_table, rel_table):
    idxs = [jnp.asarray(x, jnp.int32) for x in
            (pos_head, pos_rel, pos_tail, pos_head_exp, pos_rel_exp, pos_tail_exp)]
    return _gather6(*idxs, entity_table, rel_table)
